# Initial kernel scaffold; baseline (speedup 1.0000x reference)
import jax
import jax.numpy as jnp
from jax.experimental import pallas as pl


def kernel(x, edge_index, W_proj, b_proj, Wl0, bl0, Wr0, Wl1, bl1, Wr1):
    raise NotImplementedError("write your pallas kernel here")


# speedup vs baseline: 5.2264x; 5.2264x over previous
import functools

import jax
import jax.numpy as jnp
from jax import lax
from jax.experimental import pallas as pl
from jax.experimental.pallas import tpu as pltpu
from jax.experimental.pallas import tpu_sc as plsc

N = 10000
E = 320000
D = 128
NC = 2
NS = 16
NW = NC * NS
EPW = E // NW
CHUNK = 128
NCHUNK = -(-EPW // CHUNK)
EPW_PAD = NCHUNK * CHUNK
ACC_ROWS = 10240
RPT = ACC_ROWS // NS


def _sc_aggregate(with_deg: bool):
    mesh = plsc.VectorSubcoreMesh(core_axis_name="c", subcore_axis_name="s")
    out_type = [jax.ShapeDtypeStruct((NC, ACC_ROWS, D), jnp.float32)]
    scratch = [
        pltpu.VMEM((NCHUNK, CHUNK), jnp.int32),
        pltpu.VMEM((NCHUNK, CHUNK), jnp.int32),
        pltpu.VMEM((CHUNK, D), jnp.float32),
        pltpu.VMEM_SHARED((ACC_ROWS, D), jnp.float32),
        pltpu.SemaphoreType.DMA,
    ]
    if with_deg:
        out_type.append(jax.ShapeDtypeStruct((NC, ACC_ROWS), jnp.float32))
        scratch += [
            pltpu.VMEM((CHUNK,), jnp.float32),
            pltpu.VMEM_SHARED((ACC_ROWS,), jnp.float32),
        ]

    def body(p_hbm, srcb_hbm, dstb_hbm, agg_out, *rest):
        if with_deg:
            deg_out, src_v, dst_v, rows_v, acc_sh, sem, ones_v, deg_sh = rest
        else:
            src_v, dst_v, rows_v, acc_sh, sem = rest
        c = lax.axis_index("c")
        s = lax.axis_index("s")
        wid = c * NS + s

        pltpu.sync_copy(srcb_hbm.at[wid], src_v)
        pltpu.sync_copy(dstb_hbm.at[wid], dst_v)

        zero16 = jnp.zeros((16,), jnp.float32)
        one16 = jnp.ones((16,), jnp.float32)

        def fill(i, _):
            for k in range(D // 16):
                rows_v[i, pl.ds(k * 16, 16)] = zero16
            return 0

        lax.fori_loop(0, CHUNK, fill, 0)
        if with_deg:
            def fill1(i, _):
                ones_v[pl.ds(i * 16, 16)] = one16
                return 0
            lax.fori_loop(0, CHUNK // 16, fill1, 0)

        for k in range(RPT // CHUNK):
            pltpu.sync_copy(rows_v, acc_sh.at[pl.ds(s * RPT + k * CHUNK, CHUNK)])
        if with_deg:
            for k in range(RPT // CHUNK):
                pltpu.sync_copy(rows_v.at[0],
                                deg_sh.at[pl.ds(s * RPT + k * CHUNK, CHUNK)])
        plsc.subcore_barrier()

        def step(j, _):
            pltpu.async_copy(p_hbm.at[src_v.at[j]], rows_v, sem).wait()
            pltpu.sync_copy(rows_v, acc_sh.at[dst_v.at[j]], add=True)
            if with_deg:
                pltpu.sync_copy(ones_v, deg_sh.at[dst_v.at[j]], add=True)
            return 0

        lax.fori_loop(0, NCHUNK, step, 0)
        plsc.subcore_barrier()

        base = s * RPT
        pltpu.sync_copy(acc_sh.at[pl.ds(base, RPT)],
                        agg_out.at[c, pl.ds(base, RPT)])
        if with_deg:
            pltpu.sync_copy(deg_sh.at[pl.ds(base, RPT)],
                            deg_out.at[c, pl.ds(base, RPT)])

    return pl.kernel(body, out_type=out_type, mesh=mesh, scratch_types=scratch,
                     name="sc_agg_deg" if with_deg else "sc_agg")


_sc_agg_deg = _sc_aggregate(True)
_sc_agg = _sc_aggregate(False)


_ROWS_BLK = 2048
_GRID = -(-N // _ROWS_BLK)


def _dot_t(a, w):
    return lax.dot_general(a, w, (((1,), (1,)), ((), ())),
                           preferred_element_type=jnp.float32)


def _tc_proj_body(x_ref, wp_ref, bp_ref, wl_ref, wr_ref, bl_ref, p_ref, r_ref):
    h = _dot_t(x_ref[...], wp_ref[...]) + bp_ref[...]
    p_ref[...] = _dot_t(h, wl_ref[...])
    r_ref[...] = _dot_t(h, wr_ref[...]) + bl_ref[...]


def _tc_mid_body(agg_ref, deg_ref, r_ref, wl_ref, wr_ref, bl_ref,
                 p_ref, r1_ref):
    deg = (deg_ref[0, :] + deg_ref[1, :]).reshape(_ROWS_BLK, 1)
    inv = 1.0 / jnp.maximum(deg, 1.0)
    h = jnp.maximum((agg_ref[0] + agg_ref[1]) * inv + r_ref[...], 0.0)
    p_ref[...] = _dot_t(h, wl_ref[...])
    r1_ref[...] = _dot_t(h, wr_ref[...]) + bl_ref[...]


def _tc_out_body(agg_ref, deg_ref, r_ref, o_ref):
    deg = (deg_ref[0, :] + deg_ref[1, :]).reshape(_ROWS_BLK, 1)
    inv = 1.0 / jnp.maximum(deg, 1.0)
    o_ref[...] = jnp.maximum((agg_ref[0] + agg_ref[1]) * inv + r_ref[...], 0.0)


_row_spec = pl.BlockSpec((_ROWS_BLK, D), lambda i: (i, 0))
_w_spec = pl.BlockSpec((D, D), lambda i: (0, 0))
_b_spec = pl.BlockSpec((1, D), lambda i: (0, 0))
_agg_spec = pl.BlockSpec((NC, _ROWS_BLK, D), lambda i: (0, i, 0))
_deg_spec = pl.BlockSpec((NC, _ROWS_BLK), lambda i: (0, i))

_tc_proj = pl.pallas_call(
    _tc_proj_body,
    grid=(_GRID,),
    in_specs=[_row_spec, _w_spec, _b_spec, _w_spec, _w_spec, _b_spec],
    out_specs=[_row_spec, _row_spec],
    out_shape=[jax.ShapeDtypeStruct((N, D), jnp.float32)] * 2,
)

_tc_mid = pl.pallas_call(
    _tc_mid_body,
    grid=(_GRID,),
    in_specs=[_agg_spec, _deg_spec, _row_spec, _w_spec, _w_spec, _b_spec],
    out_specs=[_row_spec, _row_spec],
    out_shape=[jax.ShapeDtypeStruct((N, D), jnp.float32)] * 2,
)

_tc_out = pl.pallas_call(
    _tc_out_body,
    grid=(_GRID,),
    in_specs=[_agg_spec, _deg_spec, _row_spec],
    out_specs=_row_spec,
    out_shape=jax.ShapeDtypeStruct((N, D), jnp.float32),
)


def kernel(x, edge_index, W_proj, b_proj, Wl0, bl0, Wr0, Wl1, bl1, Wr1):
    src = edge_index[0].reshape(NW, EPW)
    dst = edge_index[1].reshape(NW, EPW)
    pad = EPW_PAD - EPW
    srcb = jnp.pad(src, ((0, 0), (0, pad))).reshape(NW, NCHUNK, CHUNK)
    dstb = jnp.pad(dst, ((0, 0), (0, pad)), constant_values=N).reshape(
        NW, NCHUNK, CHUNK)

    bp = b_proj.reshape(1, D)
    bl0r = bl0.reshape(1, D)
    bl1r = bl1.reshape(1, D)

    p0, r0 = _tc_proj(x, W_proj, bp, Wl0, Wr0, bl0r)
    agg0, degp = _sc_agg_deg(p0, srcb, dstb)
    p1, r1 = _tc_mid(agg0, degp, r0, Wl1, Wr1, bl1r)
    (agg1,) = _sc_agg(p1, srcb, dstb)
    out = _tc_out(agg1, degp, r1)
    return out
